# baseline (speedup 1.0000x reference)
import functools

import jax
import jax.numpy as jnp
from jax import lax
from jax.experimental import pallas as pl
from jax.experimental.pallas import tpu as pltpu
from jax.experimental.pallas import tpu_sc as plsc

B, S, D = 4, 4096, 1024
NC, NS = 2, 16
NW = NC * NS
PPW = S // NW
CH = 8
NCH = PPW // CH
NSET = 3
NBUF = NSET * B
LANES = 16
NCOL = D // LANES
QCOL = 16


def _emb_body(tokens_hbm, tok_table_hbm, pos_table_hbm, out_hbm, idx_v,
              *rest):
    tokbufs = rest[0:NBUF]
    posbufs = rest[NBUF:NBUF + 2]
    tok_sems = rest[NBUF + 2:2 * NBUF + 2]
    pos_sems = rest[2 * NBUF + 2:2 * NBUF + 4]
    wb_sems = rest[2 * NBUF + 4:3 * NBUF + 4]
    idx_sem = rest[3 * NBUF + 4]

    wid = lax.axis_index("s") * NC + lax.axis_index("c")
    p0 = wid * PPW

    idx_cps = [
        pltpu.async_copy(tokens_hbm.at[b, pl.ds(p0, PPW)],
                         idx_v.at[pl.ds(b * PPW, PPW)], idx_sem)
        for b in range(B)
    ]

    def buf_k(ci, b):
        return (ci % NSET) * B + b

    def tok_pair(ci, b):
        k = buf_k(ci, b)
        src = tok_table_hbm.at[idx_v.at[pl.ds(b * PPW + ci * CH, CH)]]
        return src, tokbufs[k], tok_sems[k]

    def pos_pair(ci):
        return (pos_table_hbm.at[pl.ds(p0 + ci * CH, CH)], posbufs[ci % 2],
                pos_sems[ci % 2])

    def wb_pair(ci, b):
        k = buf_k(ci, b)
        return (tokbufs[k], out_hbm.at[pl.ds(b * S + p0 + ci * CH, CH)],
                wb_sems[k])

    def add_chunk(ci):
        toks = [tokbufs[buf_k(ci, b)] for b in range(B)]
        pos = posbufs[ci % 2]
        nq = NCOL // QCOL

        def q_body(i, c2):
            r = i // nq
            cbase = (i % nq) * (QCOL * LANES)
            vs = [pos[r, pl.ds(cbase + c * LANES, LANES)]
                  for c in range(QCOL)]
            for tok in toks:
                for c in range(QCOL):
                    plsc.addupdate(
                        tok.at[r, pl.ds(cbase + c * LANES, LANES)], vs[c])
            return c2

        lax.fori_loop(0, CH * nq, q_body, 0)

    src, dst, sem = pos_pair(0)
    pltpu.async_copy(src, dst, sem)
    for cp in idx_cps:
        cp.wait()
    for b in range(B):
        src, dst, sem = tok_pair(0, b)
        pltpu.async_copy(src, dst, sem)

    for ci in range(NCH):
        if ci >= 2:
            for b in range(B):
                src, dst, sem = wb_pair(ci - 2, b)
                pltpu.make_async_copy(src, dst, sem).wait()
        if ci + 1 < NCH:
            for b in range(B):
                src, dst, sem = tok_pair(ci + 1, b)
                pltpu.async_copy(src, dst, sem)
            src, dst, sem = pos_pair(ci + 1)
            pltpu.async_copy(src, dst, sem)
        src, dst, sem = pos_pair(ci)
        pltpu.make_async_copy(src, dst, sem).wait()
        for b in range(B):
            src, dst, sem = tok_pair(ci, b)
            pltpu.make_async_copy(src, dst, sem).wait()
        add_chunk(ci)
        for b in range(B):
            src, dst, sem = wb_pair(ci, b)
            pltpu.async_copy(src, dst, sem)

    for ci in (NCH - 2, NCH - 1):
        for b in range(B):
            src, dst, sem = wb_pair(ci, b)
            pltpu.make_async_copy(src, dst, sem).wait()


@jax.jit
def _emb(tokens, token_table, pos_table):
    mesh = plsc.VectorSubcoreMesh(core_axis_name="c", subcore_axis_name="s")
    scratch = [pltpu.VMEM((B * PPW,), jnp.int32)]
    scratch += [pltpu.VMEM((CH, D), jnp.float32) for _ in range(NBUF + 2)]
    scratch += [pltpu.SemaphoreType.DMA for _ in range(2 * NBUF + 3)]
    kern = functools.partial(
        pl.kernel,
        mesh=mesh,
        out_type=jax.ShapeDtypeStruct((B * S, D), jnp.float32),
        scratch_types=scratch,
    )(_emb_body)
    return kern(tokens, token_table, pos_table)


def kernel(tokens, token_table, pos_table):
    out = _emb(tokens.astype(jnp.int32), token_table, pos_table)
    return out.reshape(B, S, D)


# speedup vs baseline: 1.0099x; 1.0099x over previous
import functools

import jax
import jax.numpy as jnp
from jax import lax
from jax.experimental import pallas as pl
from jax.experimental.pallas import tpu as pltpu
from jax.experimental.pallas import tpu_sc as plsc

B, S, D = 4, 4096, 1024
NC, NS = 2, 16
NW = NC * NS
PPW = S // NW
CH = 8
NCH = PPW // CH
NSLOT = NCH * B
NBUF = 8
AHEAD = 5
GROUP = 8
NGRP = NSLOT // GROUP
LANES = 16


def _emb_body(tokens_hbm, tok_table_hbm, pos_table_hbm, out_hbm, idx_v,
              *rest):
    tokbufs = rest[0:NBUF]
    posbufs = rest[NBUF:NBUF + 2]
    tok_sems = rest[NBUF + 2:2 * NBUF + 2]
    pos_sems = rest[2 * NBUF + 2:2 * NBUF + 4]
    wb_sems = rest[2 * NBUF + 4:3 * NBUF + 4]
    idx_sem = rest[3 * NBUF + 4]

    wid = lax.axis_index("s") * NC + lax.axis_index("c")
    p0 = wid * PPW

    idx_cps = [
        pltpu.async_copy(tokens_hbm.at[b, pl.ds(p0, PPW)],
                         idx_v.at[pl.ds(b * PPW, PPW)], idx_sem)
        for b in range(B)
    ]

    def tok_pair(ci, b, k):
        src = tok_table_hbm.at[idx_v.at[pl.ds(b * PPW + ci * CH, CH)]]
        return src, tokbufs[k], tok_sems[k]

    def issue_tok(ci, b, k):
        src, dst, sem = tok_pair(ci, b, k)
        pltpu.async_copy(src, dst, sem)

    def wait_tok(ci, b, k):
        src, dst, sem = tok_pair(ci, b, k)
        pltpu.make_async_copy(src, dst, sem).wait()

    def pos_pair(ci, j):
        return (pos_table_hbm.at[pl.ds(p0 + ci * CH, CH)], posbufs[j],
                pos_sems[j])

    def issue_pos(ci, j):
        src, dst, sem = pos_pair(ci, j)
        pltpu.async_copy(src, dst, sem)

    def wait_pos(ci, j):
        src, dst, sem = pos_pair(ci, j)
        pltpu.make_async_copy(src, dst, sem).wait()

    def wb_pair(ci, b, k):
        return (tokbufs[k], out_hbm.at[pl.ds(b * S + p0 + ci * CH, CH)],
                wb_sems[k])

    def issue_wb(ci, b, k):
        src, dst, sem = wb_pair(ci, b, k)
        pltpu.async_copy(src, dst, sem)

    def drain_wb(ci, b, k):
        src, dst, sem = wb_pair(ci, b, k)
        pltpu.make_async_copy(src, dst, sem).wait()

    ncol = D // LANES
    nhalf = ncol // 2

    def add_chunk(k, j):
        tok, pos = tokbufs[k], posbufs[j]

        def half_body(i, c2):
            r = i >> 1
            cbase = (i & 1) * (nhalf * LANES)
            prev = pos[r, pl.ds(cbase, LANES)]
            for col in range(1, nhalf):
                cur = pos[r, pl.ds(cbase + col * LANES, LANES)]
                plsc.addupdate(
                    tok.at[r, pl.ds(cbase + (col - 1) * LANES, LANES)], prev)
                prev = cur
            plsc.addupdate(
                tok.at[r, pl.ds(cbase + (nhalf - 1) * LANES, LANES)], prev)
            return c2

        lax.fori_loop(0, 2 * CH, half_body, 0)

    def group(ci0, first, last):
        for k in range(GROUP):
            ci, b = ci0 + k // B, k % B
            if not (first and k < AHEAD):
                cid = ci0 + (k - AHEAD) // B
                drain_wb(cid, (k - AHEAD) % B, (k - AHEAD) % NBUF)
            if not (last and k >= GROUP - AHEAD):
                cii = ci0 + (k + AHEAD) // B
                issue_tok(cii, (k + AHEAD) % B, (k + AHEAD) % NBUF)
            if b == 0:
                if k // B == 0:
                    if not first:
                        issue_pos(ci + 1, 1)
                    wait_pos(ci, 0)
                else:
                    if not last:
                        issue_pos(ci + 1, 0)
                    wait_pos(ci, 1)
            wait_tok(ci, b, k)
            add_chunk(k, (k // B) % 2)
            issue_wb(ci, b, k)

    issue_pos(0, 0)
    issue_pos(1, 1)
    for cp in idx_cps:
        cp.wait()
    for t in range(AHEAD):
        issue_tok(t // B, t % B, t)

    group(0, first=True, last=False)

    def body(g, carry):
        group(2 * g, first=False, last=False)
        return carry

    lax.fori_loop(1, NGRP - 1, body, 0)

    group(2 * (NGRP - 1), first=False, last=True)

    for t in range(NSLOT - AHEAD, NSLOT):
        ci, b, k = t // B, t % B, t % NBUF
        drain_wb(ci, b, k)


@jax.jit
def _emb(tokens, token_table, pos_table):
    mesh = plsc.VectorSubcoreMesh(core_axis_name="c", subcore_axis_name="s")
    scratch = [pltpu.VMEM((B * PPW,), jnp.int32)]
    scratch += [pltpu.VMEM((CH, D), jnp.float32) for _ in range(NBUF + 2)]
    scratch += [pltpu.SemaphoreType.DMA for _ in range(2 * NBUF + 3)]
    kern = functools.partial(
        pl.kernel,
        mesh=mesh,
        out_type=jax.ShapeDtypeStruct((B * S, D), jnp.float32),
        scratch_types=scratch,
    )(_emb_body)
    return kern(tokens, token_table, pos_table)


def kernel(tokens, token_table, pos_table):
    out = _emb(tokens.astype(jnp.int32), token_table, pos_table)
    return out.reshape(B, S, D)
